# baseline (speedup 1.0000x reference)
import functools

import jax
import jax.numpy as jnp
from jax import lax
from jax.experimental import pallas as pl
from jax.experimental.pallas import tpu as pltpu
from jax.experimental.pallas import tpu_sc as plsc

N = 10000
NP = 10240
E = 160000
EPT = E // 16
CH = 80
NCH = EPT // CH
CCH = 80
CNCH = EPT // CCH
G = 16
F32 = jnp.float32

_sc_mesh = plsc.VectorSubcoreMesh(core_axis_name="c", subcore_axis_name="s")


@functools.partial(
    pl.kernel,
    mesh=_sc_mesh,
    out_type=jax.ShapeDtypeStruct((2, NP), F32),
    scratch_types=[
        pltpu.VMEM((NCH, CH), jnp.int32),
        pltpu.VMEM((CH,), F32),
        pltpu.VMEM((640,), F32),
        pltpu.VMEM_SHARED((NP,), F32),
    ],
)
def _deg_sc(dstd, deg_out, idx_v, ones_v, zbuf, deg_sh):
    c = lax.axis_index("c")
    s = lax.axis_index("s")
    for k in range(640 // 16):
        zbuf[pl.ds(k * 16, 16)] = jnp.zeros((16,), F32)
    for k in range(CH // 16):
        ones_v[pl.ds(k * 16, 16)] = jnp.ones((16,), F32)
    pltpu.sync_copy(zbuf, deg_sh.at[pl.ds(s * 640, 640)])
    pltpu.sync_copy(dstd.at[c, s], idx_v)
    plsc.subcore_barrier()

    def body(j, carry):
        pltpu.sync_copy(ones_v, deg_sh.at[idx_v.at[j]], add=True)
        return carry

    lax.fori_loop(0, NCH, body, 0)
    plsc.subcore_barrier()
    pltpu.sync_copy(deg_sh.at[pl.ds(s * 640, 640)],
                    deg_out.at[c, pl.ds(s * 640, 640)])


@functools.partial(
    pl.kernel,
    mesh=_sc_mesh,
    out_type=jax.ShapeDtypeStruct((2, NP, 128), F32),
    scratch_types=[
        pltpu.VMEM((EPT,), jnp.int32),
        pltpu.VMEM((CNCH, CCH), jnp.int32),
        pltpu.VMEM((CCH, 128), F32),
        pltpu.VMEM((CCH, 128), F32),
        pltpu.SemaphoreType.DMA,
        pltpu.SemaphoreType.DMA,
        pltpu.SemaphoreType.DMA,
        pltpu.SemaphoreType.DMA,
        pltpu.VMEM_SHARED((NP, 128), F32),
    ],
)
def _conv_sc(hp_flat, src_pre, dst_pre, acc_out, src_v, dst_v, rows0, rows1,
             gsem0, gsem1, ssem0, ssem1, acc_sh):
    c = lax.axis_index("c")
    s = lax.axis_index("s")
    pltpu.sync_copy(src_pre.at[c, s], src_v)
    pltpu.sync_copy(dst_pre.at[s], dst_v)
    pltpu.async_copy(hp_flat.at[src_v.at[pl.ds(0, CCH)]], rows0, gsem0)
    pltpu.async_copy(hp_flat.at[src_v.at[pl.ds(CCH, CCH)]], rows1, gsem1)
    pltpu.sync_copy(hp_flat.at[pl.ds(c * NP + s * 640, 640)],
                    acc_sh.at[pl.ds(s * 640, 640)])
    plsc.subcore_barrier()

    def body(i, carry):
        k = 2 * i
        pltpu.make_async_copy(
            hp_flat.at[src_v.at[pl.ds(k * CCH, CCH)]], rows0, gsem0).wait()
        hs0 = pltpu.async_copy(rows0, acc_sh.at[dst_v.at[k]], ssem0, add=True)
        pltpu.make_async_copy(
            hp_flat.at[src_v.at[pl.ds((k + 1) * CCH, CCH)]], rows1,
            gsem1).wait()
        hs1 = pltpu.async_copy(rows1, acc_sh.at[dst_v.at[k + 1]], ssem1,
                               add=True)
        hs0.wait()
        pltpu.async_copy(
            hp_flat.at[src_v.at[pl.ds((k + 2) * CCH, CCH)]], rows0, gsem0)
        hs1.wait()

        @pl.when(i < CNCH // 2 - 1)
        def _fire():
            pltpu.async_copy(
                hp_flat.at[src_v.at[pl.ds((k + 3) * CCH, CCH)]], rows1, gsem1)
        return carry

    lax.fori_loop(0, CNCH // 2, body, 0)
    pltpu.make_async_copy(
        hp_flat.at[src_v.at[pl.ds((CNCH - 1) * CCH, CCH)]], rows0,
        gsem0).wait()
    pltpu.sync_copy(rows0, acc_sh.at[dst_v.at[CNCH - 1]], add=True)
    plsc.subcore_barrier()
    pltpu.sync_copy(acc_sh.at[pl.ds(s * 640, 640)],
                    acc_out.at[c, pl.ds(s * 640, 640)])


def _mm1_body(x_ref, deg_ref, w_ref, hp_ref, dinv_ref):
    dinv = lax.rsqrt(deg_ref[...] + 1.0)
    h = jnp.dot(x_ref[...], w_ref[...], preferred_element_type=F32)
    hp_ref[0] = dinv * h[:, :128]
    hp_ref[1] = dinv * h[:, 128:]
    dinv_ref[...] = dinv


def _mm23_body(acc_ref, dinv_ref, b_ref, w_ref, hp_ref):
    dinv = dinv_ref[...]
    x0 = jnp.maximum(dinv * acc_ref[0] + b_ref[0, :128], 0.0)
    x1 = jnp.maximum(dinv * acc_ref[1] + b_ref[0, 128:], 0.0)
    h = (jnp.dot(x0, w_ref[0], preferred_element_type=F32)
         + jnp.dot(x1, w_ref[1], preferred_element_type=F32))
    hp_ref[0] = dinv * h[:, :128]
    hp_ref[1] = dinv * h[:, 128:]


def _mm1(x_pad, deg, W):
    degb = jnp.broadcast_to(deg[:, None], (NP, 128))
    return pl.pallas_call(
        _mm1_body,
        grid=(16,),
        in_specs=[
            pl.BlockSpec((640, 256), lambda i: (i, 0)),
            pl.BlockSpec((640, 128), lambda i: (i, 0)),
            pl.BlockSpec((256, 256), lambda i: (0, 0)),
        ],
        out_specs=[
            pl.BlockSpec((2, 640, 128), lambda i: (0, i, 0)),
            pl.BlockSpec((640, 128), lambda i: (i, 0)),
        ],
        out_shape=[
            jax.ShapeDtypeStruct((2, NP, 128), F32),
            jax.ShapeDtypeStruct((NP, 128), F32),
        ],
    )(x_pad, degb, W)


def _mm23(acc, dinvb, b, W):
    return pl.pallas_call(
        _mm23_body,
        grid=(16,),
        in_specs=[
            pl.BlockSpec((2, 640, 128), lambda i: (0, i, 0)),
            pl.BlockSpec((640, 128), lambda i: (i, 0)),
            pl.BlockSpec((1, 256), lambda i: (0, 0)),
            pl.BlockSpec((2, 128, 256), lambda i: (0, 0, 0)),
        ],
        out_specs=pl.BlockSpec((2, 640, 128), lambda i: (0, i, 0)),
        out_shape=jax.ShapeDtypeStruct((2, NP, 128), F32),
    )(acc, dinvb, b.reshape(1, 256), W.reshape(2, 128, 256))


def _head_body(acc1_ref, dinv1_ref, bt1_ref, acc2_ref, dinv2_ref, bt2_ref,
               b3_ref, l1w_ref, l1b_ref, l2w_ref, l2b_ref, out_ref,
               p1_scr, c1_scr, p2_scr, c2_scr):
    i = pl.program_id(0)

    @pl.when(i == 0)
    def _init():
        p1_scr[...] = jnp.zeros((G, 256), F32)
        c1_scr[...] = jnp.zeros((G, 128), F32)
        p2_scr[...] = jnp.zeros((G, 256), F32)
        c2_scr[...] = jnp.zeros((G, 128), F32)

    def accum(acc_ref, dinv_ref, bt_ref, p_scr, c_scr):
        dinv = dinv_ref[...]
        x0 = dinv * acc_ref[0] + b3_ref[0, :128]
        x1 = dinv * acc_ref[1] + b3_ref[0, 128:]
        bt = bt_ref[0]
        P = (lax.broadcasted_iota(jnp.int32, (G, 640), 0) == bt).astype(F32)
        p_scr[:, :128] += jnp.dot(P, x0, preferred_element_type=F32)
        p_scr[:, 128:] += jnp.dot(P, x1, preferred_element_type=F32)
        c_scr[...] += jnp.sum(P, axis=1, keepdims=True)

    accum(acc1_ref, dinv1_ref, bt1_ref, p1_scr, c1_scr)
    accum(acc2_ref, dinv2_ref, bt2_ref, p2_scr, c2_scr)

    @pl.when(i == 15)
    def _fin():
        m1 = p1_scr[...] / jnp.maximum(c1_scr[:, :1], 1.0)
        m2 = p2_scr[...] / jnp.maximum(c2_scr[:, :1], 1.0)
        g1 = jnp.dot(m1, l1w_ref[...], preferred_element_type=F32) + l1b_ref[0]
        g2 = jnp.dot(m2, l1w_ref[...], preferred_element_type=F32) + l1b_ref[0]
        out_ref[...] = (jnp.dot(g1, l2w_ref[0], preferred_element_type=F32)
                        + jnp.dot(g2, l2w_ref[1], preferred_element_type=F32)
                        + l2b_ref[0])


def _head(acc1, dinvb1, bt1, acc2, dinvb2, bt2, b3, l1w, l1b, l2w, l2b):
    return pl.pallas_call(
        _head_body,
        grid=(16,),
        in_specs=[
            pl.BlockSpec((2, 640, 128), lambda i: (0, i, 0)),
            pl.BlockSpec((640, 128), lambda i: (i, 0)),
            pl.BlockSpec((1, 1, 640), lambda i: (i, 0, 0)),
            pl.BlockSpec((2, 640, 128), lambda i: (0, i, 0)),
            pl.BlockSpec((640, 128), lambda i: (i, 0)),
            pl.BlockSpec((1, 1, 640), lambda i: (i, 0, 0)),
            pl.BlockSpec((1, 256), lambda i: (0, 0)),
            pl.BlockSpec((256, 128), lambda i: (0, 0)),
            pl.BlockSpec((1, 128), lambda i: (0, 0)),
            pl.BlockSpec((2, 128, 1), lambda i: (0, 0, 0)),
            pl.BlockSpec((1, 1), lambda i: (0, 0)),
        ],
        out_specs=pl.BlockSpec((G, 1), lambda i: (0, 0)),
        out_shape=jax.ShapeDtypeStruct((G, 1), F32),
        scratch_shapes=[
            pltpu.VMEM((G, 256), F32),
            pltpu.VMEM((G, 128), F32),
            pltpu.VMEM((G, 256), F32),
            pltpu.VMEM((G, 128), F32),
        ],
    )(acc1, dinvb1, bt1, acc2, dinvb2, bt2, b3.reshape(1, 256), l1w,
      l1b.reshape(1, 128), l2w.reshape(2, 128, 1), l2b.reshape(1, 1))


def kernel(x1, edge_index1, batch1, x2, edge_index2, batch2,
           W1, b1, W2, b2, W3, b3, lin1_W, lin1_b, lin2_W, lin2_b):
    ei1 = edge_index1.astype(jnp.int32)
    ei2 = edge_index2.astype(jnp.int32)
    src1 = ei1[0].reshape(16, EPT)
    dst1 = ei1[1].reshape(16, CNCH, CCH)
    src2 = ei2[0].reshape(16, EPT)
    dst2 = ei2[1].reshape(16, CNCH, CCH)
    src1_pre = jnp.stack([src1, src1 + NP])
    src2_pre = jnp.stack([src2, src2 + NP])
    dstd = jnp.stack([ei1[1].reshape(16, NCH, CH),
                      ei2[1].reshape(16, NCH, CH)])

    deg = _deg_sc(dstd)

    x1p = jnp.pad(x1, ((0, NP - N), (0, 0)))
    x2p = jnp.pad(x2, ((0, NP - N), (0, 0)))
    bt1 = jnp.pad(batch1.astype(jnp.int32), (0, NP - N),
                  constant_values=G).reshape(16, 1, 640)
    bt2 = jnp.pad(batch2.astype(jnp.int32), (0, NP - N),
                  constant_values=G).reshape(16, 1, 640)

    def branch(x_pad, deg_b, src_pre, dst_r):
        hp, dinvb = _mm1(x_pad, deg_b, W1)
        acc = _conv_sc(hp.reshape(2 * NP, 128), src_pre, dst_r)
        hp = _mm23(acc, dinvb, b1, W2)
        acc = _conv_sc(hp.reshape(2 * NP, 128), src_pre, dst_r)
        hp = _mm23(acc, dinvb, b2, W3)
        acc = _conv_sc(hp.reshape(2 * NP, 128), src_pre, dst_r)
        return acc, dinvb

    acc1, dinvb1 = branch(x1p, deg[0], src1_pre, dst1)
    acc2, dinvb2 = branch(x2p, deg[1], src2_pre, dst2)

    return _head(acc1, dinvb1, bt1, acc2, dinvb2, bt2,
                 b3, lin1_W, lin1_b, lin2_W, lin2_b)


# speedup vs baseline: 1.2489x; 1.2489x over previous
import functools

import jax
import jax.numpy as jnp
from jax import lax
from jax.experimental import pallas as pl
from jax.experimental.pallas import tpu as pltpu
from jax.experimental.pallas import tpu_sc as plsc

N = 10000
NP = 10240
E = 160000
EPT = E // 16
CH = 80
NCH = EPT // CH
CCH = 80
CNCH = EPT // CCH
G = 16
F32 = jnp.float32

_sc_mesh = plsc.VectorSubcoreMesh(core_axis_name="c", subcore_axis_name="s")


@functools.partial(
    pl.kernel,
    mesh=_sc_mesh,
    out_type=jax.ShapeDtypeStruct((2, NP), F32),
    scratch_types=[
        pltpu.VMEM((NCH, CH), jnp.int32),
        pltpu.VMEM((CH,), F32),
        pltpu.VMEM((640,), F32),
        pltpu.VMEM_SHARED((NP,), F32),
    ],
)
def _deg_sc(dstd, deg_out, idx_v, ones_v, zbuf, deg_sh):
    c = lax.axis_index("c")
    s = lax.axis_index("s")
    for k in range(640 // 16):
        zbuf[pl.ds(k * 16, 16)] = jnp.zeros((16,), F32)
    for k in range(CH // 16):
        ones_v[pl.ds(k * 16, 16)] = jnp.ones((16,), F32)
    pltpu.sync_copy(zbuf, deg_sh.at[pl.ds(s * 640, 640)])
    pltpu.sync_copy(dstd.at[c, s], idx_v)
    plsc.subcore_barrier()

    def body(j, carry):
        pltpu.sync_copy(ones_v, deg_sh.at[idx_v.at[j]], add=True)
        return carry

    lax.fori_loop(0, NCH, body, 0)
    plsc.subcore_barrier()
    pltpu.sync_copy(deg_sh.at[pl.ds(s * 640, 640)],
                    deg_out.at[c, pl.ds(s * 640, 640)])


@functools.partial(
    pl.kernel,
    mesh=_sc_mesh,
    out_type=jax.ShapeDtypeStruct((2, NP, 128), F32),
    scratch_types=[
        pltpu.VMEM((EPT,), jnp.int32),
        pltpu.VMEM((CNCH, CCH), jnp.int32),
        pltpu.VMEM((CCH, 128), F32),
        pltpu.VMEM((CCH, 128), F32),
        pltpu.SemaphoreType.DMA,
        pltpu.SemaphoreType.DMA,
        pltpu.VMEM_SHARED((NP, 128), F32),
    ],
)
def _conv_sc(hp_flat, src_pre, dst_pre, acc_out, src_v, dst_v, rows0, rows1,
             gsem0, gsem1, acc_sh):
    c = lax.axis_index("c")
    s = lax.axis_index("s")
    for q in range(CCH):
        for l in range(128 // 16):
            rows0[q, pl.ds(l * 16, 16)] = jnp.zeros((16,), F32)
    for q in range(640 // CCH):
        pltpu.sync_copy(rows0, acc_sh.at[pl.ds(s * 640 + q * CCH, CCH)])
    pltpu.sync_copy(src_pre.at[c, s], src_v)
    pltpu.sync_copy(dst_pre.at[s], dst_v)
    pltpu.async_copy(hp_flat.at[src_v.at[pl.ds(0, CCH)]], rows0, gsem0)
    plsc.subcore_barrier()

    def body(i, carry):
        k = 2 * i
        h1 = pltpu.async_copy(
            hp_flat.at[src_v.at[pl.ds((k + 1) * CCH, CCH)]], rows1, gsem1)
        pltpu.make_async_copy(
            hp_flat.at[src_v.at[pl.ds(k * CCH, CCH)]], rows0, gsem0).wait()
        pltpu.sync_copy(rows0, acc_sh.at[dst_v.at[k]], add=True)
        pltpu.async_copy(
            hp_flat.at[src_v.at[pl.ds((k + 2) * CCH, CCH)]], rows0, gsem0)
        h1.wait()
        pltpu.sync_copy(rows1, acc_sh.at[dst_v.at[k + 1]], add=True)
        return carry

    lax.fori_loop(0, CNCH // 2, body, 0)
    pltpu.make_async_copy(
        hp_flat.at[src_v.at[pl.ds((CNCH - 1) * CCH, CCH)]], rows0,
        gsem0).wait()
    pltpu.sync_copy(rows0, acc_sh.at[dst_v.at[CNCH - 1]], add=True)
    plsc.subcore_barrier()
    pltpu.sync_copy(acc_sh.at[pl.ds(s * 640, 640)],
                    acc_out.at[c, pl.ds(s * 640, 640)])


def _mm1_body(x_ref, deg_ref, w_ref, hp_ref, dinv_ref):
    dinv = lax.rsqrt(deg_ref[...] + 1.0)
    h = jnp.dot(x_ref[...], w_ref[...], preferred_element_type=F32)
    hp_ref[0] = dinv * h[:, :128]
    hp_ref[1] = dinv * h[:, 128:]
    dinv_ref[...] = dinv


def _mm23_body(acc_ref, hpp_ref, dinv_ref, b_ref, w_ref, hp_ref):
    dinv = dinv_ref[...]
    x0 = jnp.maximum(dinv * (acc_ref[0] + hpp_ref[0]) + b_ref[0, :128], 0.0)
    x1 = jnp.maximum(dinv * (acc_ref[1] + hpp_ref[1]) + b_ref[0, 128:], 0.0)
    h = (jnp.dot(x0, w_ref[0], preferred_element_type=F32)
         + jnp.dot(x1, w_ref[1], preferred_element_type=F32))
    hp_ref[0] = dinv * h[:, :128]
    hp_ref[1] = dinv * h[:, 128:]


def _mm1(x_pad, deg, W):
    degb = jnp.broadcast_to(deg[:, None], (NP, 128))
    return pl.pallas_call(
        _mm1_body,
        grid=(16,),
        in_specs=[
            pl.BlockSpec((640, 256), lambda i: (i, 0)),
            pl.BlockSpec((640, 128), lambda i: (i, 0)),
            pl.BlockSpec((256, 256), lambda i: (0, 0)),
        ],
        out_specs=[
            pl.BlockSpec((2, 640, 128), lambda i: (0, i, 0)),
            pl.BlockSpec((640, 128), lambda i: (i, 0)),
        ],
        out_shape=[
            jax.ShapeDtypeStruct((2, NP, 128), F32),
            jax.ShapeDtypeStruct((NP, 128), F32),
        ],
    )(x_pad, degb, W)


def _mm23(acc, hp_prev, dinvb, b, W):
    return pl.pallas_call(
        _mm23_body,
        grid=(16,),
        in_specs=[
            pl.BlockSpec((2, 640, 128), lambda i: (0, i, 0)),
            pl.BlockSpec((2, 640, 128), lambda i: (0, i, 0)),
            pl.BlockSpec((640, 128), lambda i: (i, 0)),
            pl.BlockSpec((1, 256), lambda i: (0, 0)),
            pl.BlockSpec((2, 128, 256), lambda i: (0, 0, 0)),
        ],
        out_specs=pl.BlockSpec((2, 640, 128), lambda i: (0, i, 0)),
        out_shape=jax.ShapeDtypeStruct((2, NP, 128), F32),
    )(acc, hp_prev, dinvb, b.reshape(1, 256), W.reshape(2, 128, 256))


def _head_body(acc1_ref, hpp1_ref, dinv1_ref, bt1_ref,
               acc2_ref, hpp2_ref, dinv2_ref, bt2_ref,
               b3_ref, l1w_ref, l1b_ref, l2w_ref, l2b_ref, out_ref,
               p1_scr, c1_scr, p2_scr, c2_scr):
    i = pl.program_id(0)

    @pl.when(i == 0)
    def _init():
        p1_scr[...] = jnp.zeros((G, 256), F32)
        c1_scr[...] = jnp.zeros((G, 128), F32)
        p2_scr[...] = jnp.zeros((G, 256), F32)
        c2_scr[...] = jnp.zeros((G, 128), F32)

    def accum(acc_ref, hpp_ref, dinv_ref, bt_ref, p_scr, c_scr):
        dinv = dinv_ref[...]
        x0 = dinv * (acc_ref[0] + hpp_ref[0]) + b3_ref[0, :128]
        x1 = dinv * (acc_ref[1] + hpp_ref[1]) + b3_ref[0, 128:]
        bt = bt_ref[0]
        P = (lax.broadcasted_iota(jnp.int32, (G, 640), 0) == bt).astype(F32)
        p_scr[:, :128] += jnp.dot(P, x0, preferred_element_type=F32)
        p_scr[:, 128:] += jnp.dot(P, x1, preferred_element_type=F32)
        c_scr[...] += jnp.sum(P, axis=1, keepdims=True)

    accum(acc1_ref, hpp1_ref, dinv1_ref, bt1_ref, p1_scr, c1_scr)
    accum(acc2_ref, hpp2_ref, dinv2_ref, bt2_ref, p2_scr, c2_scr)

    @pl.when(i == 15)
    def _fin():
        m1 = p1_scr[...] / jnp.maximum(c1_scr[:, :1], 1.0)
        m2 = p2_scr[...] / jnp.maximum(c2_scr[:, :1], 1.0)
        g1 = jnp.dot(m1, l1w_ref[...], preferred_element_type=F32) + l1b_ref[0]
        g2 = jnp.dot(m2, l1w_ref[...], preferred_element_type=F32) + l1b_ref[0]
        out_ref[...] = (jnp.dot(g1, l2w_ref[0], preferred_element_type=F32)
                        + jnp.dot(g2, l2w_ref[1], preferred_element_type=F32)
                        + l2b_ref[0])


def _head(acc1, hp1, dinvb1, bt1, acc2, hp2, dinvb2, bt2,
          b3, l1w, l1b, l2w, l2b):
    return pl.pallas_call(
        _head_body,
        grid=(16,),
        in_specs=[
            pl.BlockSpec((2, 640, 128), lambda i: (0, i, 0)),
            pl.BlockSpec((2, 640, 128), lambda i: (0, i, 0)),
            pl.BlockSpec((640, 128), lambda i: (i, 0)),
            pl.BlockSpec((1, 1, 640), lambda i: (i, 0, 0)),
            pl.BlockSpec((2, 640, 128), lambda i: (0, i, 0)),
            pl.BlockSpec((2, 640, 128), lambda i: (0, i, 0)),
            pl.BlockSpec((640, 128), lambda i: (i, 0)),
            pl.BlockSpec((1, 1, 640), lambda i: (i, 0, 0)),
            pl.BlockSpec((1, 256), lambda i: (0, 0)),
            pl.BlockSpec((256, 128), lambda i: (0, 0)),
            pl.BlockSpec((1, 128), lambda i: (0, 0)),
            pl.BlockSpec((2, 128, 1), lambda i: (0, 0, 0)),
            pl.BlockSpec((1, 1), lambda i: (0, 0)),
        ],
        out_specs=pl.BlockSpec((G, 1), lambda i: (0, 0)),
        out_shape=jax.ShapeDtypeStruct((G, 1), F32),
        scratch_shapes=[
            pltpu.VMEM((G, 256), F32),
            pltpu.VMEM((G, 128), F32),
            pltpu.VMEM((G, 256), F32),
            pltpu.VMEM((G, 128), F32),
        ],
    )(acc1, hp1, dinvb1, bt1, acc2, hp2, dinvb2, bt2, b3.reshape(1, 256),
      l1w, l1b.reshape(1, 128), l2w.reshape(2, 128, 1), l2b.reshape(1, 1))


def kernel(x1, edge_index1, batch1, x2, edge_index2, batch2,
           W1, b1, W2, b2, W3, b3, lin1_W, lin1_b, lin2_W, lin2_b):
    ei1 = edge_index1.astype(jnp.int32)
    ei2 = edge_index2.astype(jnp.int32)
    src1 = ei1[0].reshape(16, EPT)
    dst1 = ei1[1].reshape(16, CNCH, CCH)
    src2 = ei2[0].reshape(16, EPT)
    dst2 = ei2[1].reshape(16, CNCH, CCH)
    src1_pre = jnp.stack([src1, src1 + NP])
    src2_pre = jnp.stack([src2, src2 + NP])
    dstd = jnp.stack([ei1[1].reshape(16, NCH, CH),
                      ei2[1].reshape(16, NCH, CH)])

    deg = _deg_sc(dstd)

    x1p = jnp.pad(x1, ((0, NP - N), (0, 0)))
    x2p = jnp.pad(x2, ((0, NP - N), (0, 0)))
    bt1 = jnp.pad(batch1.astype(jnp.int32), (0, NP - N),
                  constant_values=G).reshape(16, 1, 640)
    bt2 = jnp.pad(batch2.astype(jnp.int32), (0, NP - N),
                  constant_values=G).reshape(16, 1, 640)

    def branch(x_pad, deg_b, src_pre, dst_r):
        hp1, dinvb = _mm1(x_pad, deg_b, W1)
        acc = _conv_sc(hp1.reshape(2 * NP, 128), src_pre, dst_r)
        hp2 = _mm23(acc, hp1, dinvb, b1, W2)
        acc = _conv_sc(hp2.reshape(2 * NP, 128), src_pre, dst_r)
        hp3 = _mm23(acc, hp2, dinvb, b2, W3)
        acc = _conv_sc(hp3.reshape(2 * NP, 128), src_pre, dst_r)
        return acc, hp3, dinvb

    acc1, hp31, dinvb1 = branch(x1p, deg[0], src1_pre, dst1)
    acc2, hp32, dinvb2 = branch(x2p, deg[1], src2_pre, dst2)

    return _head(acc1, hp31, dinvb1, bt1, acc2, hp32, dinvb2, bt2,
                 b3, lin1_W, lin1_b, lin2_W, lin2_b)


# speedup vs baseline: 1.2529x; 1.0032x over previous
import functools

import jax
import jax.numpy as jnp
from jax import lax
from jax.experimental import pallas as pl
from jax.experimental.pallas import tpu as pltpu
from jax.experimental.pallas import tpu_sc as plsc

N = 10000
NP = 10240
E = 160000
EPT = E // 16
CH = 80
NCH = EPT // CH
CCH = 80
CNCH = EPT // CCH
G = 16
F32 = jnp.float32

_sc_mesh = plsc.VectorSubcoreMesh(core_axis_name="c", subcore_axis_name="s")


@functools.partial(
    pl.kernel,
    mesh=_sc_mesh,
    out_type=jax.ShapeDtypeStruct((2, NP), F32),
    scratch_types=[
        pltpu.VMEM((NCH, CH), jnp.int32),
        pltpu.VMEM((CH,), F32),
        pltpu.VMEM((640,), F32),
        pltpu.VMEM_SHARED((NP,), F32),
    ],
)
def _deg_sc(dstd, deg_out, idx_v, ones_v, zbuf, deg_sh):
    c = lax.axis_index("c")
    s = lax.axis_index("s")
    for k in range(640 // 16):
        zbuf[pl.ds(k * 16, 16)] = jnp.zeros((16,), F32)
    for k in range(CH // 16):
        ones_v[pl.ds(k * 16, 16)] = jnp.ones((16,), F32)
    pltpu.sync_copy(zbuf, deg_sh.at[pl.ds(s * 640, 640)])
    pltpu.sync_copy(dstd.at[c, s], idx_v)
    plsc.subcore_barrier()

    def body(j, carry):
        pltpu.sync_copy(ones_v, deg_sh.at[idx_v.at[j]], add=True)
        return carry

    lax.fori_loop(0, NCH, body, 0)
    plsc.subcore_barrier()
    pltpu.sync_copy(deg_sh.at[pl.ds(s * 640, 640)],
                    deg_out.at[c, pl.ds(s * 640, 640)])


@functools.partial(
    pl.kernel,
    mesh=_sc_mesh,
    out_type=jax.ShapeDtypeStruct((2, NP, 128), F32),
    scratch_types=[
        pltpu.VMEM((EPT,), jnp.int32),
        pltpu.VMEM((CNCH, CCH), jnp.int32),
        pltpu.VMEM((CCH, 128), F32),
        pltpu.VMEM((CCH, 128), F32),
        pltpu.SemaphoreType.DMA,
        pltpu.SemaphoreType.DMA,
        pltpu.VMEM_SHARED((NP, 128), F32),
    ],
)
def _conv_sc(hp_flat, src_pre, dst_pre, acc_out, src_v, dst_v, rows0, rows1,
             gsem0, gsem1, acc_sh):
    c = lax.axis_index("c")
    s = lax.axis_index("s")
    pltpu.sync_copy(src_pre.at[c, s], src_v)
    pltpu.sync_copy(dst_pre.at[s], dst_v)
    pltpu.async_copy(hp_flat.at[src_v.at[pl.ds(0, CCH)]], rows0, gsem0)
    pltpu.sync_copy(hp_flat.at[pl.ds(c * NP + s * 640, 640)],
                    acc_sh.at[pl.ds(s * 640, 640)])
    plsc.subcore_barrier()

    def body(i, carry):
        k = 2 * i
        h1 = pltpu.async_copy(
            hp_flat.at[src_v.at[pl.ds((k + 1) * CCH, CCH)]], rows1, gsem1)
        pltpu.make_async_copy(
            hp_flat.at[src_v.at[pl.ds(k * CCH, CCH)]], rows0, gsem0).wait()
        pltpu.sync_copy(rows0, acc_sh.at[dst_v.at[k]], add=True)
        pltpu.async_copy(
            hp_flat.at[src_v.at[pl.ds((k + 2) * CCH, CCH)]], rows0, gsem0)
        h1.wait()
        pltpu.sync_copy(rows1, acc_sh.at[dst_v.at[k + 1]], add=True)
        return carry

    lax.fori_loop(0, CNCH // 2, body, 0)
    pltpu.make_async_copy(
        hp_flat.at[src_v.at[pl.ds((CNCH - 1) * CCH, CCH)]], rows0,
        gsem0).wait()
    pltpu.sync_copy(rows0, acc_sh.at[dst_v.at[CNCH - 1]], add=True)
    plsc.subcore_barrier()
    pltpu.sync_copy(acc_sh.at[pl.ds(s * 640, 640)],
                    acc_out.at[c, pl.ds(s * 640, 640)])


def _mm1_body(x_ref, deg_ref, w_ref, hp_ref, dinv_ref):
    dinv = lax.rsqrt(deg_ref[...] + 1.0)
    h = jnp.dot(x_ref[...], w_ref[...], preferred_element_type=F32)
    hp_ref[0] = dinv * h[:, :128]
    hp_ref[1] = dinv * h[:, 128:]
    dinv_ref[...] = dinv


def _mm23_body(acc_ref, dinv_ref, b_ref, w_ref, hp_ref):
    dinv = dinv_ref[...]
    x0 = jnp.maximum(dinv * acc_ref[0] + b_ref[0, :128], 0.0)
    x1 = jnp.maximum(dinv * acc_ref[1] + b_ref[0, 128:], 0.0)
    h = (jnp.dot(x0, w_ref[0], preferred_element_type=F32)
         + jnp.dot(x1, w_ref[1], preferred_element_type=F32))
    hp_ref[0] = dinv * h[:, :128]
    hp_ref[1] = dinv * h[:, 128:]


def _mm1(x_pad, deg, W):
    degb = jnp.broadcast_to(deg[:, None], (NP, 128))
    return pl.pallas_call(
        _mm1_body,
        grid=(16,),
        in_specs=[
            pl.BlockSpec((640, 256), lambda i: (i, 0)),
            pl.BlockSpec((640, 128), lambda i: (i, 0)),
            pl.BlockSpec((256, 256), lambda i: (0, 0)),
        ],
        out_specs=[
            pl.BlockSpec((2, 640, 128), lambda i: (0, i, 0)),
            pl.BlockSpec((640, 128), lambda i: (i, 0)),
        ],
        out_shape=[
            jax.ShapeDtypeStruct((2, NP, 128), F32),
            jax.ShapeDtypeStruct((NP, 128), F32),
        ],
    )(x_pad, degb, W)


def _mm23(acc, dinvb, b, W):
    return pl.pallas_call(
        _mm23_body,
        grid=(16,),
        in_specs=[
            pl.BlockSpec((2, 640, 128), lambda i: (0, i, 0)),
            pl.BlockSpec((640, 128), lambda i: (i, 0)),
            pl.BlockSpec((1, 256), lambda i: (0, 0)),
            pl.BlockSpec((2, 128, 256), lambda i: (0, 0, 0)),
        ],
        out_specs=pl.BlockSpec((2, 640, 128), lambda i: (0, i, 0)),
        out_shape=jax.ShapeDtypeStruct((2, NP, 128), F32),
    )(acc, dinvb, b.reshape(1, 256), W.reshape(2, 128, 256))


def _head_body(acc1_ref, dinv1_ref, bt1_ref, acc2_ref, dinv2_ref, bt2_ref,
               b3_ref, l1w_ref, l1b_ref, l2w_ref, l2b_ref, out_ref,
               p1_scr, c1_scr, p2_scr, c2_scr):
    i = pl.program_id(0)

    @pl.when(i == 0)
    def _init():
        p1_scr[...] = jnp.zeros((G, 256), F32)
        c1_scr[...] = jnp.zeros((G, 128), F32)
        p2_scr[...] = jnp.zeros((G, 256), F32)
        c2_scr[...] = jnp.zeros((G, 128), F32)

    def accum(acc_ref, dinv_ref, bt_ref, p_scr, c_scr):
        dinv = dinv_ref[...]
        x0 = dinv * acc_ref[0] + b3_ref[0, :128]
        x1 = dinv * acc_ref[1] + b3_ref[0, 128:]
        bt = bt_ref[0]
        P = (lax.broadcasted_iota(jnp.int32, (G, 640), 0) == bt).astype(F32)
        p_scr[:, :128] += jnp.dot(P, x0, preferred_element_type=F32)
        p_scr[:, 128:] += jnp.dot(P, x1, preferred_element_type=F32)
        c_scr[...] += jnp.sum(P, axis=1, keepdims=True)

    accum(acc1_ref, dinv1_ref, bt1_ref, p1_scr, c1_scr)
    accum(acc2_ref, dinv2_ref, bt2_ref, p2_scr, c2_scr)

    @pl.when(i == 15)
    def _fin():
        m1 = p1_scr[...] / jnp.maximum(c1_scr[:, :1], 1.0)
        m2 = p2_scr[...] / jnp.maximum(c2_scr[:, :1], 1.0)
        g1 = jnp.dot(m1, l1w_ref[...], preferred_element_type=F32) + l1b_ref[0]
        g2 = jnp.dot(m2, l1w_ref[...], preferred_element_type=F32) + l1b_ref[0]
        out_ref[...] = (jnp.dot(g1, l2w_ref[0], preferred_element_type=F32)
                        + jnp.dot(g2, l2w_ref[1], preferred_element_type=F32)
                        + l2b_ref[0])


def _head(acc1, dinvb1, bt1, acc2, dinvb2, bt2, b3, l1w, l1b, l2w, l2b):
    return pl.pallas_call(
        _head_body,
        grid=(16,),
        in_specs=[
            pl.BlockSpec((2, 640, 128), lambda i: (0, i, 0)),
            pl.BlockSpec((640, 128), lambda i: (i, 0)),
            pl.BlockSpec((1, 1, 640), lambda i: (i, 0, 0)),
            pl.BlockSpec((2, 640, 128), lambda i: (0, i, 0)),
            pl.BlockSpec((640, 128), lambda i: (i, 0)),
            pl.BlockSpec((1, 1, 640), lambda i: (i, 0, 0)),
            pl.BlockSpec((1, 256), lambda i: (0, 0)),
            pl.BlockSpec((256, 128), lambda i: (0, 0)),
            pl.BlockSpec((1, 128), lambda i: (0, 0)),
            pl.BlockSpec((2, 128, 1), lambda i: (0, 0, 0)),
            pl.BlockSpec((1, 1), lambda i: (0, 0)),
        ],
        out_specs=pl.BlockSpec((G, 1), lambda i: (0, 0)),
        out_shape=jax.ShapeDtypeStruct((G, 1), F32),
        scratch_shapes=[
            pltpu.VMEM((G, 256), F32),
            pltpu.VMEM((G, 128), F32),
            pltpu.VMEM((G, 256), F32),
            pltpu.VMEM((G, 128), F32),
        ],
    )(acc1, dinvb1, bt1, acc2, dinvb2, bt2, b3.reshape(1, 256), l1w,
      l1b.reshape(1, 128), l2w.reshape(2, 128, 1), l2b.reshape(1, 1))


def kernel(x1, edge_index1, batch1, x2, edge_index2, batch2,
           W1, b1, W2, b2, W3, b3, lin1_W, lin1_b, lin2_W, lin2_b):
    ei1 = edge_index1.astype(jnp.int32)
    ei2 = edge_index2.astype(jnp.int32)
    src1 = ei1[0].reshape(16, EPT)
    dst1 = ei1[1].reshape(16, CNCH, CCH)
    src2 = ei2[0].reshape(16, EPT)
    dst2 = ei2[1].reshape(16, CNCH, CCH)
    src1_pre = jnp.stack([src1, src1 + NP])
    src2_pre = jnp.stack([src2, src2 + NP])
    dstd = jnp.stack([ei1[1].reshape(16, NCH, CH),
                      ei2[1].reshape(16, NCH, CH)])

    deg = _deg_sc(dstd)

    x1p = jnp.pad(x1, ((0, NP - N), (0, 0)))
    x2p = jnp.pad(x2, ((0, NP - N), (0, 0)))
    bt1 = jnp.pad(batch1.astype(jnp.int32), (0, NP - N),
                  constant_values=G).reshape(16, 1, 640)
    bt2 = jnp.pad(batch2.astype(jnp.int32), (0, NP - N),
                  constant_values=G).reshape(16, 1, 640)

    def branch(x_pad, deg_b, src_pre, dst_r):
        hp, dinvb = _mm1(x_pad, deg_b, W1)
        acc = _conv_sc(hp.reshape(2 * NP, 128), src_pre, dst_r)
        hp = _mm23(acc, dinvb, b1, W2)
        acc = _conv_sc(hp.reshape(2 * NP, 128), src_pre, dst_r)
        hp = _mm23(acc, dinvb, b2, W3)
        acc = _conv_sc(hp.reshape(2 * NP, 128), src_pre, dst_r)
        return acc, dinvb

    acc1, dinvb1 = branch(x1p, deg[0], src1_pre, dst1)
    acc2, dinvb2 = branch(x2p, deg[1], src2_pre, dst2)

    return _head(acc1, dinvb1, bt1, acc2, dinvb2, bt2,
                 b3, lin1_W, lin1_b, lin2_W, lin2_b)


# speedup vs baseline: 1.2554x; 1.0019x over previous
import functools

import jax
import jax.numpy as jnp
from jax import lax
from jax.experimental import pallas as pl
from jax.experimental.pallas import tpu as pltpu
from jax.experimental.pallas import tpu_sc as plsc

N = 10000
NP = 10240
E = 160000
EPT = E // 16
CH = 80
NCH = EPT // CH
CCH = 80
CNCH = EPT // CCH
G = 16
F32 = jnp.float32

_sc_mesh = plsc.VectorSubcoreMesh(core_axis_name="c", subcore_axis_name="s")


@functools.partial(
    pl.kernel,
    mesh=_sc_mesh,
    out_type=jax.ShapeDtypeStruct((2, NP), F32),
    scratch_types=[
        pltpu.VMEM((NCH, CH), jnp.int32),
        pltpu.VMEM((CH,), F32),
        pltpu.VMEM((640,), F32),
        pltpu.VMEM_SHARED((NP,), F32),
    ],
)
def _deg_sc(dstd, deg_out, idx_v, ones_v, zbuf, deg_sh):
    c = lax.axis_index("c")
    s = lax.axis_index("s")
    for k in range(640 // 16):
        zbuf[pl.ds(k * 16, 16)] = jnp.zeros((16,), F32)
    for k in range(CH // 16):
        ones_v[pl.ds(k * 16, 16)] = jnp.ones((16,), F32)
    pltpu.sync_copy(zbuf, deg_sh.at[pl.ds(s * 640, 640)])
    pltpu.sync_copy(dstd.at[c, s], idx_v)
    plsc.subcore_barrier()

    def body(j, carry):
        pltpu.sync_copy(ones_v, deg_sh.at[idx_v.at[j]], add=True)
        return carry

    lax.fori_loop(0, NCH, body, 0)
    plsc.subcore_barrier()
    pltpu.sync_copy(deg_sh.at[pl.ds(s * 640, 640)],
                    deg_out.at[c, pl.ds(s * 640, 640)])


@functools.partial(
    pl.kernel,
    mesh=_sc_mesh,
    out_type=jax.ShapeDtypeStruct((2, NP, 128), F32),
    scratch_types=[
        pltpu.VMEM((EPT,), jnp.int32),
        pltpu.VMEM((CNCH, CCH), jnp.int32),
        pltpu.VMEM((CCH, 128), F32),
        pltpu.VMEM((CCH, 128), F32),
        pltpu.SemaphoreType.DMA,
        pltpu.SemaphoreType.DMA,
        pltpu.VMEM_SHARED((NP, 128), F32),
    ],
)
def _conv_sc(hp_flat, src_pre, dst_pre, acc_out, src_v, dst_v, rows0, rows1,
             gsem0, gsem1, acc_sh):
    c = lax.axis_index("c")
    s = lax.axis_index("s")
    pltpu.sync_copy(src_pre.at[c, s], src_v)
    pltpu.sync_copy(dst_pre.at[s], dst_v)
    pltpu.async_copy(hp_flat.at[src_v.at[pl.ds(0, CCH)]], rows0, gsem0)
    pltpu.sync_copy(hp_flat.at[pl.ds(c * NP + s * 640, 640)],
                    acc_sh.at[pl.ds(s * 640, 640)])
    plsc.subcore_barrier()

    def body(i, carry):
        k = 2 * i
        h1 = pltpu.async_copy(
            hp_flat.at[src_v.at[pl.ds((k + 1) * CCH, CCH)]], rows1, gsem1)
        pltpu.make_async_copy(
            hp_flat.at[src_v.at[pl.ds(k * CCH, CCH)]], rows0, gsem0).wait()
        pltpu.sync_copy(rows0, acc_sh.at[dst_v.at[k]], add=True)
        pltpu.async_copy(
            hp_flat.at[src_v.at[pl.ds((k + 2) * CCH, CCH)]], rows0, gsem0)
        h1.wait()
        pltpu.sync_copy(rows1, acc_sh.at[dst_v.at[k + 1]], add=True)
        return carry

    lax.fori_loop(0, CNCH // 2, body, 0)
    pltpu.make_async_copy(
        hp_flat.at[src_v.at[pl.ds((CNCH - 1) * CCH, CCH)]], rows0,
        gsem0).wait()
    pltpu.sync_copy(rows0, acc_sh.at[dst_v.at[CNCH - 1]], add=True)
    plsc.subcore_barrier()
    pltpu.sync_copy(acc_sh.at[pl.ds(s * 640, 640)],
                    acc_out.at[c, pl.ds(s * 640, 640)])


def _mm1_body(x_ref, deg_ref, w_ref, hp_ref, dinv_ref):
    dinv = lax.rsqrt(deg_ref[...] + 1.0)
    h = jnp.dot(x_ref[...], w_ref[...], preferred_element_type=F32)
    hp_ref[0] = dinv * h[:, :128]
    hp_ref[1] = dinv * h[:, 128:]
    dinv_ref[...] = dinv


def _mm23_body(acc_ref, dinv_ref, b_ref, w_ref, hp_ref):
    dinv = dinv_ref[...]
    x0 = jnp.maximum(dinv * acc_ref[0] + b_ref[0, :128], 0.0)
    x1 = jnp.maximum(dinv * acc_ref[1] + b_ref[0, 128:], 0.0)
    h = (jnp.dot(x0, w_ref[0], preferred_element_type=F32)
         + jnp.dot(x1, w_ref[1], preferred_element_type=F32))
    hp_ref[0] = dinv * h[:, :128]
    hp_ref[1] = dinv * h[:, 128:]


def _mm1(x_pad, deg, W):
    degb = jnp.broadcast_to(deg[:, None], (NP, 128))
    return pl.pallas_call(
        _mm1_body,
        grid=(16,),
        in_specs=[
            pl.BlockSpec((640, 256), lambda i: (i, 0)),
            pl.BlockSpec((640, 128), lambda i: (i, 0)),
            pl.BlockSpec((256, 256), lambda i: (0, 0)),
        ],
        out_specs=[
            pl.BlockSpec((2, 640, 128), lambda i: (0, i, 0)),
            pl.BlockSpec((640, 128), lambda i: (i, 0)),
        ],
        out_shape=[
            jax.ShapeDtypeStruct((2, NP, 128), F32),
            jax.ShapeDtypeStruct((NP, 128), F32),
        ],
    )(x_pad, degb, W)


def _mm23(acc, dinvb, b, W):
    return pl.pallas_call(
        _mm23_body,
        grid=(16,),
        in_specs=[
            pl.BlockSpec((2, 640, 128), lambda i: (0, i, 0)),
            pl.BlockSpec((640, 128), lambda i: (i, 0)),
            pl.BlockSpec((1, 256), lambda i: (0, 0)),
            pl.BlockSpec((2, 128, 256), lambda i: (0, 0, 0)),
        ],
        out_specs=pl.BlockSpec((2, 640, 128), lambda i: (0, i, 0)),
        out_shape=jax.ShapeDtypeStruct((2, NP, 128), F32),
    )(acc, dinvb, b.reshape(1, 256), W.reshape(2, 128, 256))


def _head_body(acc1_ref, dinv1_ref, bt1_ref, acc2_ref, dinv2_ref, bt2_ref,
               b3_ref, l1w_ref, l1b_ref, l2w_ref, l2b_ref, out_ref,
               p1_scr, c1_scr, p2_scr, c2_scr):
    i = pl.program_id(0)

    @pl.when(i == 0)
    def _init():
        p1_scr[...] = jnp.zeros((G, 256), F32)
        c1_scr[...] = jnp.zeros((G, 128), F32)
        p2_scr[...] = jnp.zeros((G, 256), F32)
        c2_scr[...] = jnp.zeros((G, 128), F32)

    def accum(acc_ref, dinv_ref, bt_ref, p_scr, c_scr):
        dinv = dinv_ref[...]
        x0 = dinv * acc_ref[0] + b3_ref[0, :128]
        x1 = dinv * acc_ref[1] + b3_ref[0, 128:]
        bt = bt_ref[0]
        P = (lax.broadcasted_iota(jnp.int32, (G, 640), 0) == bt).astype(F32)
        p_scr[:, :128] += jnp.dot(P, x0, preferred_element_type=F32)
        p_scr[:, 128:] += jnp.dot(P, x1, preferred_element_type=F32)
        c_scr[...] += jnp.sum(P, axis=1, keepdims=True)

    accum(acc1_ref, dinv1_ref, bt1_ref, p1_scr, c1_scr)
    accum(acc2_ref, dinv2_ref, bt2_ref, p2_scr, c2_scr)

    @pl.when(i == 15)
    def _fin():
        m1 = p1_scr[...] / jnp.maximum(c1_scr[:, :1], 1.0)
        m2 = p2_scr[...] / jnp.maximum(c2_scr[:, :1], 1.0)
        g1 = jnp.dot(m1, l1w_ref[...], preferred_element_type=F32) + l1b_ref[0]
        g2 = jnp.dot(m2, l1w_ref[...], preferred_element_type=F32) + l1b_ref[0]
        out_ref[...] = (jnp.dot(g1, l2w_ref[0], preferred_element_type=F32)
                        + jnp.dot(g2, l2w_ref[1], preferred_element_type=F32)
                        + l2b_ref[0])


def _head(acc1, dinvb1, bt1, acc2, dinvb2, bt2, b3, l1w, l1b, l2w, l2b):
    return pl.pallas_call(
        _head_body,
        grid=(16,),
        in_specs=[
            pl.BlockSpec((2, 640, 128), lambda i: (0, i, 0)),
            pl.BlockSpec((640, 128), lambda i: (i, 0)),
            pl.BlockSpec((1, 1, 640), lambda i: (i, 0, 0)),
            pl.BlockSpec((2, 640, 128), lambda i: (0, i, 0)),
            pl.BlockSpec((640, 128), lambda i: (i, 0)),
            pl.BlockSpec((1, 1, 640), lambda i: (i, 0, 0)),
            pl.BlockSpec((1, 256), lambda i: (0, 0)),
            pl.BlockSpec((256, 128), lambda i: (0, 0)),
            pl.BlockSpec((1, 128), lambda i: (0, 0)),
            pl.BlockSpec((2, 128, 1), lambda i: (0, 0, 0)),
            pl.BlockSpec((1, 1), lambda i: (0, 0)),
        ],
        out_specs=pl.BlockSpec((G, 1), lambda i: (0, 0)),
        out_shape=jax.ShapeDtypeStruct((G, 1), F32),
        scratch_shapes=[
            pltpu.VMEM((G, 256), F32),
            pltpu.VMEM((G, 128), F32),
            pltpu.VMEM((G, 256), F32),
            pltpu.VMEM((G, 128), F32),
        ],
    )(acc1, dinvb1, bt1, acc2, dinvb2, bt2, b3.reshape(1, 256), l1w,
      l1b.reshape(1, 128), l2w.reshape(2, 128, 1), l2b.reshape(1, 1))


def kernel(x1, edge_index1, batch1, x2, edge_index2, batch2,
           W1, b1, W2, b2, W3, b3, lin1_W, lin1_b, lin2_W, lin2_b):
    ei1 = edge_index1.astype(jnp.int32)
    ei2 = edge_index2.astype(jnp.int32)
    src1 = ei1[0].reshape(16, EPT)
    dst1 = ei1[1].reshape(16, CNCH, CCH)
    src2 = ei2[0].reshape(16, EPT)
    dst2 = ei2[1].reshape(16, CNCH, CCH)
    src1_pre = jnp.stack([src1, src1 + NP])
    src2_pre = jnp.stack([src2, src2 + NP])
    dstd = jnp.stack([ei1[1].reshape(16, NCH, CH),
                      ei2[1].reshape(16, NCH, CH)])

    deg = _deg_sc(dstd)

    x1p = jnp.pad(x1, ((0, NP - N), (0, 0)))
    x2p = jnp.pad(x2, ((0, NP - N), (0, 0)))
    bt1 = jnp.pad(batch1.astype(jnp.int32), (0, NP - N),
                  constant_values=G).reshape(16, 1, 640)
    bt2 = jnp.pad(batch2.astype(jnp.int32), (0, NP - N),
                  constant_values=G).reshape(16, 1, 640)

    hp1a, dinvb1 = _mm1(x1p, deg[0], W1)
    hp1b, dinvb2 = _mm1(x2p, deg[1], W1)
    acc1a = _conv_sc(hp1a.reshape(2 * NP, 128), src1_pre, dst1)
    acc1b = _conv_sc(hp1b.reshape(2 * NP, 128), src2_pre, dst2)
    hp2a = _mm23(acc1a, dinvb1, b1, W2)
    hp2b = _mm23(acc1b, dinvb2, b1, W2)
    acc2a = _conv_sc(hp2a.reshape(2 * NP, 128), src1_pre, dst1)
    acc2b = _conv_sc(hp2b.reshape(2 * NP, 128), src2_pre, dst2)
    hp3a = _mm23(acc2a, dinvb1, b2, W3)
    hp3b = _mm23(acc2b, dinvb2, b2, W3)
    acc1 = _conv_sc(hp3a.reshape(2 * NP, 128), src1_pre, dst1)
    acc2 = _conv_sc(hp3b.reshape(2 * NP, 128), src2_pre, dst2)

    return _head(acc1, dinvb1, bt1, acc2, dinvb2, bt2,
                 b3, lin1_W, lin1_b, lin2_W, lin2_b)
